# baseline (speedup 1.0000x reference)
import jax
import jax.numpy as jnp
from jax import lax
from jax.experimental import pallas as pl
from jax.experimental.pallas import tpu as pltpu
from jax.experimental.pallas import tpu_sc as plsc

NUM_CORES = 2
NUM_SUBCORES = 16
NUM_WORKERS = NUM_CORES * NUM_SUBCORES
LANES = 16

ROWS = 4 * 8192
D = 1024
ROWS_PER_WORKER = ROWS // NUM_WORKERS
CHUNK = 16
NCHUNKS = ROWS_PER_WORKER // CHUNK
NBUF = 4


def _body(x_hbm, keep_hbm, out_hbm, keep_v, zrow_v, spmem,
          in_sems, patch_sems, out_sems):
    sid = lax.axis_index("s")
    wid = sid * NUM_CORES + lax.axis_index("c")
    base = wid * ROWS_PER_WORKER

    pltpu.sync_copy(keep_hbm.at[pl.ds(base, ROWS_PER_WORKER)], keep_v)

    zeros = jnp.zeros((LANES,), jnp.float32)
    for j in range(D // LANES):
        zrow_v[0, pl.ds(j * LANES, LANES)] = zeros

    def in_pass(g, slot, start):
        kvec = keep_v[pl.ds(g * CHUNK, CHUNK)]
        for r in range(CHUNK):
            @pl.when(kvec[r] > 0.0)
            def _(r=r):
                d = pltpu.make_async_copy(
                    x_hbm.at[pl.ds(base + g * CHUNK + r, 1)],
                    spmem.at[sid, slot, pl.ds(r, 1)], in_sems.at[slot])
                if start:
                    d.start()
                else:
                    d.wait()

    def patch_row(slot, r, start):
        d = pltpu.make_async_copy(
            zrow_v, spmem.at[sid, slot, pl.ds(r, 1)], patch_sems.at[slot])
        if start:
            d.start()
        else:
            d.wait()

    def patch_pass(g, slot, start):
        kvec = keep_v[pl.ds(g * CHUNK, CHUNK)]

        @pl.when(g < NBUF)
        def _():
            for r in range(CHUNK):
                @pl.when(kvec[r] == 0.0)
                def _(r=r):
                    patch_row(slot, r, start)

        @pl.when(g >= NBUF)
        def _():
            pvec = keep_v[pl.ds((g - NBUF) * CHUNK, CHUNK)]
            for r in range(CHUNK):
                @pl.when((kvec[r] == 0.0) & (pvec[r] > 0.0))
                def _(r=r):
                    patch_row(slot, r, start)

    def out_desc(g, slot):
        return pltpu.make_async_copy(
            spmem.at[sid, slot],
            out_hbm.at[pl.ds(base + g * CHUNK, CHUNK)], out_sems.at[slot])

    for g0 in (0, 1):
        in_pass(g0, g0, start=True)
        patch_pass(g0, g0, start=True)

    def group_body(go, _):
        for i in range(NBUF):
            g = go * NBUF + i
            gp = g + 2
            slot_p = (i + 2) % NBUF

            @pl.when(gp < NCHUNKS)
            def _():
                @pl.when(gp >= NBUF)
                def _():
                    out_desc(gp - NBUF, slot_p).wait()
                in_pass(gp, slot_p, start=True)
                patch_pass(gp, slot_p, start=True)

            in_pass(g, i, start=False)
            patch_pass(g, i, start=False)
            out_desc(g, i).start()
        return 0

    lax.fori_loop(0, NCHUNKS // NBUF, group_body, 0)

    for g in range(NCHUNKS - NBUF, NCHUNKS):
        out_desc(g, g % NBUF).wait()


@jax.jit
def _masked_zero(x2d, keep):
    mesh = plsc.VectorSubcoreMesh(
        core_axis_name="c", subcore_axis_name="s",
        num_cores=NUM_CORES, num_subcores=NUM_SUBCORES)
    return pl.kernel(
        _body,
        out_type=jax.ShapeDtypeStruct((ROWS, D), jnp.float32),
        mesh=mesh,
        scratch_types=[
            pltpu.VMEM((ROWS_PER_WORKER,), jnp.float32),
            pltpu.VMEM((1, D), jnp.float32),
            pltpu.VMEM_SHARED((NUM_SUBCORES, NBUF, CHUNK, D), jnp.float32),
            pltpu.SemaphoreType.DMA((NBUF,)),
            pltpu.SemaphoreType.DMA((NBUF,)),
            pltpu.SemaphoreType.DMA((NBUF,)),
        ],
    )(x2d, keep)


def kernel(x, x_pad_mask):
    x2d = x.reshape(ROWS, D)
    keep = jnp.where(x_pad_mask.reshape(ROWS), 0.0, 1.0).astype(jnp.float32)
    out = _masked_zero(x2d, keep)
    return out.reshape(x.shape)


# speedup vs baseline: 1.0380x; 1.0380x over previous
import jax
import jax.numpy as jnp
from jax import lax
from jax.experimental import pallas as pl
from jax.experimental.pallas import tpu as pltpu
from jax.experimental.pallas import tpu_sc as plsc

NUM_CORES = 2
NUM_SUBCORES = 16
NUM_WORKERS = NUM_CORES * NUM_SUBCORES
LANES = 16

ROWS = 4 * 8192
D = 1024
ROWS_PER_WORKER = ROWS // NUM_WORKERS
CHUNK = 16
NCHUNKS = ROWS_PER_WORKER // CHUNK
NBUF = 4


def _body(x_hbm, keep_hbm, out_hbm, keep_v, zrow_v, spmem,
          in_sems, patch_sems, out_sems):
    sid = lax.axis_index("s")
    wid = sid * NUM_CORES + lax.axis_index("c")
    base = wid * ROWS_PER_WORKER

    pltpu.sync_copy(keep_hbm.at[pl.ds(base, ROWS_PER_WORKER)], keep_v)

    zeros = jnp.zeros((LANES,), jnp.float32)
    for j in range(D // LANES):
        zrow_v[0, pl.ds(j * LANES, LANES)] = zeros

    def in_pass(g, slot, start):
        kvec = keep_v[pl.ds(g * CHUNK, CHUNK)]
        for r in range(CHUNK):
            @pl.when(kvec[r] > 0.0)
            def _(r=r):
                d = pltpu.make_async_copy(
                    x_hbm.at[pl.ds(base + g * CHUNK + r, 1)],
                    spmem.at[sid, slot, pl.ds(r, 1)], in_sems.at[slot])
                if start:
                    d.start()
                else:
                    d.wait()

    def patch_pass(g, slot, start):
        kvec = keep_v[pl.ds(g * CHUNK, CHUNK)]
        for r in range(CHUNK):
            @pl.when(kvec[r] == 0.0)
            def _(r=r):
                d = pltpu.make_async_copy(
                    zrow_v, spmem.at[sid, slot, pl.ds(r, 1)],
                    patch_sems.at[slot])
                if start:
                    d.start()
                else:
                    d.wait()

    def out_desc(g, slot):
        return pltpu.make_async_copy(
            spmem.at[sid, slot],
            out_hbm.at[pl.ds(base + g * CHUNK, CHUNK)], out_sems.at[slot])

    for g0 in (0, 1):
        in_pass(g0, g0, start=True)
        patch_pass(g0, g0, start=True)

    def group_body(go, _):
        for i in range(NBUF):
            g = go * NBUF + i
            gp = g + 2
            slot_p = (i + 2) % NBUF

            @pl.when(gp < NCHUNKS)
            def _():
                @pl.when(gp >= NBUF)
                def _():
                    out_desc(gp - NBUF, slot_p).wait()
                in_pass(gp, slot_p, start=True)
                patch_pass(gp, slot_p, start=True)

            in_pass(g, i, start=False)
            patch_pass(g, i, start=False)
            out_desc(g, i).start()
        return 0

    lax.fori_loop(0, NCHUNKS // NBUF, group_body, 0)

    for g in range(NCHUNKS - NBUF, NCHUNKS):
        out_desc(g, g % NBUF).wait()


@jax.jit
def _masked_zero(x2d, keep):
    mesh = plsc.VectorSubcoreMesh(
        core_axis_name="c", subcore_axis_name="s",
        num_cores=NUM_CORES, num_subcores=NUM_SUBCORES)
    return pl.kernel(
        _body,
        out_type=jax.ShapeDtypeStruct((ROWS, D), jnp.float32),
        mesh=mesh,
        scratch_types=[
            pltpu.VMEM((ROWS_PER_WORKER,), jnp.float32),
            pltpu.VMEM((1, D), jnp.float32),
            pltpu.VMEM_SHARED((NUM_SUBCORES, NBUF, CHUNK, D), jnp.float32),
            pltpu.SemaphoreType.DMA((NBUF,)),
            pltpu.SemaphoreType.DMA((NBUF,)),
            pltpu.SemaphoreType.DMA((NBUF,)),
        ],
    )(x2d, keep)


def kernel(x, x_pad_mask):
    x2d = x.reshape(ROWS, D)
    keep = jnp.where(x_pad_mask.reshape(ROWS), 0.0, 1.0).astype(jnp.float32)
    out = _masked_zero(x2d, keep)
    return out.reshape(x.shape)
